# baseline (speedup 1.0000x reference)
import functools

import jax
import jax.numpy as jnp
from jax import lax
from jax.experimental import pallas as pl
from jax.experimental.pallas import tpu as pltpu
from jax.experimental.pallas import tpu_sc as plsc

BATCH = 4096
N_FIELDS = 26
EMB_DIM = 128

NUM_CORES = 2
NUM_SUBCORES = 16
NUM_WORKERS = NUM_CORES * NUM_SUBCORES
BLK = BATCH // NUM_WORKERS


def _gather_t_body(xt_hbm, table_hbm, out_hbm, idx_v, gbufs, tbufs, *sems):
    gsems, ssems = sems[:2], sems[2:]
    wid = lax.axis_index("s") * NUM_CORES + lax.axis_index("c")
    pltpu.sync_copy(xt_hbm.at[:, pl.ds(wid * BLK, BLK)], idx_v)

    def gather(f):
        return pltpu.async_copy(
            table_hbm.at[idx_v.at[f]], gbufs.at[f % 2], gsems[f % 2]
        )

    def store(f):
        return pltpu.async_copy(tbufs.at[f % 2], out_hbm.at[f, :, wid], ssems[f % 2])

    iota = lax.iota(jnp.int32, 16)
    dts = [lax.shift_right_logical(iota + d0 * 16, 3) for d0 in range(8)]
    dss = [lax.bitwise_and(iota + d0 * 16, 7) for d0 in range(8)]

    def transpose(f):
        g = gbufs.at[f % 2]
        t = tbufs.at[f % 2]

        @pl.loop(0, BLK, unroll=4)
        def _row(b):
            blv = jnp.full((16,), b, jnp.int32)
            for d0 in range(8):
                vals = g[b, pl.ds(d0 * 16, 16)]
                plsc.store_scatter(t, [dts[d0], dss[d0], blv], vals)

    gathers = [None] * N_FIELDS
    stores = [None] * N_FIELDS
    gathers[0] = gather(0)
    gathers[1] = gather(1)
    for f in range(N_FIELDS):
        gathers[f].wait()
        if f >= 2:
            stores[f - 2].wait()
        transpose(f)
        stores[f] = store(f)
        if f + 2 < N_FIELDS:
            gathers[f + 2] = gather(f + 2)
    stores[N_FIELDS - 2].wait()
    stores[N_FIELDS - 1].wait()


@jax.jit
def _sc_gather_t(xt, table):
    mesh = plsc.VectorSubcoreMesh(core_axis_name="c", subcore_axis_name="s")
    k = functools.partial(
        pl.kernel,
        out_type=jax.ShapeDtypeStruct(
            (N_FIELDS, EMB_DIM // 8, NUM_WORKERS, 8, BLK), jnp.float32
        ),
        mesh=mesh,
        scratch_types=[
            pltpu.VMEM((N_FIELDS, BLK), jnp.int32),
            pltpu.VMEM((2, BLK, EMB_DIM), jnp.float32),
            pltpu.VMEM((2, EMB_DIM // 8, 8, BLK), jnp.float32),
        ]
        + [pltpu.SemaphoreType.DMA] * 4,
        compiler_params=pltpu.CompilerParams(
            use_tc_tiling_on_sc=False, needs_layout_passes=False
        ),
    )(_gather_t_body)
    return k(xt, table)


def kernel(x, table):
    raw = _sc_gather_t(x.T, table)
    return raw.transpose(2, 4, 0, 1, 3).reshape(BATCH, N_FIELDS, 1, 8, 16)


# speedup vs baseline: 2.1880x; 2.1880x over previous
import functools

import jax
import jax.numpy as jnp
from jax import lax
from jax.experimental import pallas as pl
from jax.experimental.pallas import tpu as pltpu
from jax.experimental.pallas import tpu_sc as plsc

BATCH = 4096
N_FIELDS = 26
EMB_DIM = 128

NUM_CORES = 2
NUM_SUBCORES = 16
NUM_WORKERS = NUM_CORES * NUM_SUBCORES
BLK = BATCH // NUM_WORKERS


def _gather_t_body(xt_hbm, table_hbm, out_hbm, idx_v, gbufs, tbufs, *sems):
    gsems, ssems = sems[:2], sems[2:]
    wid = lax.axis_index("s") * NUM_CORES + lax.axis_index("c")
    pltpu.sync_copy(xt_hbm.at[:, pl.ds(wid * BLK, BLK)], idx_v)

    def gather(f):
        return pltpu.async_copy(
            table_hbm.at[idx_v.at[f]], gbufs.at[f % 2], gsems[f % 2]
        )

    def store(f):
        return pltpu.async_copy(
            tbufs.at[f % 2, :, :, pl.ds(0, BLK)], out_hbm.at[f, :, wid], ssems[f % 2]
        )

    iota = lax.iota(jnp.int32, 16)
    dts = [lax.shift_right_logical(iota + d0 * 16, 3) for d0 in range(8)]
    dss = [lax.bitwise_and(iota + d0 * 16, 7) for d0 in range(8)]

    def transpose(f):
        g = gbufs.at[f % 2]
        t = tbufs.at[f % 2]

        @pl.loop(0, BLK)
        def _row(b):
            blv = jnp.full((16,), b, jnp.int32)
            for d0 in range(8):
                vals = g[b, pl.ds(d0 * 16, 16)]
                plsc.store_scatter(t, [dts[d0], dss[d0], blv], vals)

    gathers = [None] * N_FIELDS
    stores = [None] * N_FIELDS
    gathers[0] = gather(0)
    gathers[1] = gather(1)
    for f in range(N_FIELDS):
        gathers[f].wait()
        if f >= 2:
            stores[f - 2].wait()
        transpose(f)
        stores[f] = store(f)
        if f + 2 < N_FIELDS:
            gathers[f + 2] = gather(f + 2)
    stores[N_FIELDS - 2].wait()
    stores[N_FIELDS - 1].wait()


@jax.jit
def _sc_gather_t(xt, table):
    mesh = plsc.VectorSubcoreMesh(core_axis_name="c", subcore_axis_name="s")
    k = functools.partial(
        pl.kernel,
        out_type=jax.ShapeDtypeStruct(
            (N_FIELDS, EMB_DIM // 8, NUM_WORKERS, 8, BLK), jnp.float32
        ),
        mesh=mesh,
        scratch_types=[
            pltpu.VMEM((N_FIELDS, BLK), jnp.int32),
            pltpu.VMEM((2, BLK, EMB_DIM), jnp.float32),
            pltpu.VMEM((2, EMB_DIM // 8, 8, BLK + 1), jnp.float32),
        ]
        + [pltpu.SemaphoreType.DMA] * 4,
        compiler_params=pltpu.CompilerParams(
            use_tc_tiling_on_sc=False, needs_layout_passes=False
        ),
    )(_gather_t_body)
    return k(xt, table)


def kernel(x, table):
    raw = _sc_gather_t(x.T, table)
    return raw.transpose(2, 4, 0, 1, 3).reshape(BATCH, N_FIELDS, 1, 8, 16)


# speedup vs baseline: 3.1906x; 1.4582x over previous
import functools

import jax
import jax.numpy as jnp
from jax import lax
from jax.experimental import pallas as pl
from jax.experimental.pallas import tpu as pltpu
from jax.experimental.pallas import tpu_sc as plsc

BATCH = 4096
N_FIELDS = 26
EMB_DIM = 128

NUM_CORES = 2
NUM_SUBCORES = 16
NUM_WORKERS = NUM_CORES * NUM_SUBCORES
BLK = BATCH // NUM_WORKERS


def _gather_t_body(xt_hbm, table_hbm, out_hbm, idx_v, gbufs, tbufs, *sems):
    gsems, ssems = sems[:2], sems[2:]
    wid = lax.axis_index("s") * NUM_CORES + lax.axis_index("c")
    pltpu.sync_copy(xt_hbm.at[:, pl.ds(wid * BLK, BLK)], idx_v)

    def gather(f):
        return pltpu.async_copy(
            table_hbm.at[idx_v.at[f]], gbufs.at[f % 2], gsems[f % 2]
        )

    def store(f):
        return pltpu.async_copy(
            tbufs.at[f % 2, :, :, pl.ds(0, BLK)], out_hbm.at[f, :, wid], ssems[f % 2]
        )

    iota = lax.iota(jnp.int32, 16)
    dts = [lax.shift_right_logical(iota + d0 * 16, 3) for d0 in range(8)]
    dss = [lax.bitwise_and(iota + d0 * 16, 7) for d0 in range(8)]

    def transpose(f):
        g = gbufs.at[f % 2]
        t = tbufs.at[f % 2]

        @plsc.parallel_loop(0, BLK)
        def _row(b):
            blv = jnp.full((16,), b, jnp.int32)
            vals = [g[b, pl.ds(d0 * 16, 16)] for d0 in range(8)]
            for d0 in range(8):
                plsc.store_scatter(t, [dts[d0], dss[d0], blv], vals[d0])

    gathers = [None] * N_FIELDS
    stores = [None] * N_FIELDS
    gathers[0] = gather(0)
    gathers[1] = gather(1)
    for f in range(N_FIELDS):
        gathers[f].wait()
        if f >= 2:
            stores[f - 2].wait()
        transpose(f)
        stores[f] = store(f)
        if f + 2 < N_FIELDS:
            gathers[f + 2] = gather(f + 2)
    stores[N_FIELDS - 2].wait()
    stores[N_FIELDS - 1].wait()


@jax.jit
def _sc_gather_t(xt, table):
    mesh = plsc.VectorSubcoreMesh(core_axis_name="c", subcore_axis_name="s")
    k = functools.partial(
        pl.kernel,
        out_type=jax.ShapeDtypeStruct(
            (N_FIELDS, EMB_DIM // 8, NUM_WORKERS, 8, BLK), jnp.float32
        ),
        mesh=mesh,
        scratch_types=[
            pltpu.VMEM((N_FIELDS, BLK), jnp.int32),
            pltpu.VMEM((2, BLK, EMB_DIM), jnp.float32),
            pltpu.VMEM((2, EMB_DIM // 8, 8, BLK + 1), jnp.float32),
        ]
        + [pltpu.SemaphoreType.DMA] * 4,
        compiler_params=pltpu.CompilerParams(
            use_tc_tiling_on_sc=False, needs_layout_passes=False
        ),
    )(_gather_t_body)
    return k(xt, table)


def kernel(x, table):
    raw = _sc_gather_t(x.T, table)
    return raw.transpose(2, 4, 0, 1, 3).reshape(BATCH, N_FIELDS, 1, 8, 16)
